# Initial kernel scaffold; baseline (speedup 1.0000x reference)
import jax
import jax.numpy as jnp
from jax.experimental import pallas as pl


def kernel(x, edge_index, conn, Wq, Wk, Wv, We, Aw, Bw, Ho_w, Ho_b, Eow, Eob, g1h, b1h, g1e, b1e, W1, b1, W2, b2, g2h, b2h):
    raise NotImplementedError("write your pallas kernel here")


# speedup vs baseline: 48.1903x; 48.1903x over previous
import functools

import jax
import jax.numpy as jnp
from jax import lax
from jax.experimental import pallas as pl
from jax.experimental.pallas import tpu as pltpu
from jax.experimental.pallas import tpu_sc as plsc

N = 10000
E = 320000
D = 128
H = 8
DH = 16
CLAMP = 5.0

NSC = 2
NTILE = 16
NW = NSC * NTILE
CHUNK = 128
NCHUNK = E // CHUNK
ROWS_PER_SC = 5120
ACC_ROWS = 5248
NOUT = NSC * ROWS_PER_SC
ROWS_PER_TILE = ACC_ROWS // NTILE
OUT_PER_TILE = ROWS_PER_SC // NTILE

BN = 2000
BE = 1280


def _qkv_body(x_ref, w_ref, q_ref, k_ref, v_ref):
    y = jnp.dot(x_ref[...], w_ref[...], preferred_element_type=jnp.float32)
    q_ref[...] = y[:, :D]
    k_ref[...] = y[:, D:2 * D]
    v_ref[...] = y[:, 2 * D:]


def _edge_body(conn_ref, g_ref, vs_ref, wet_ref, eowt_ref, eob_ref, g1e_ref,
               b1e_ref, amat_ref, sel_ref, bblk_ref,
               e_ref, pay_ref, wden_ref):
    conn = conn_ref[...]
    eh = jnp.dot(conn, wet_ref[...], preferred_element_type=jnp.float32)
    ew = eh[:, :D]
    eb = eh[:, D:]
    c1 = g_ref[...] * ew
    c2 = jnp.sign(c1) * jnp.sqrt(jnp.abs(c1))
    cact = jnp.maximum(c2 + eb, 0.0)
    ev = conn + jnp.dot(cact, eowt_ref[...],
                        preferred_element_type=jnp.float32) + eob_ref[...]
    m = jnp.mean(ev, axis=-1, keepdims=True)
    vv = jnp.mean((ev - m) ** 2, axis=-1, keepdims=True)
    e_ref[...] = (ev - m) * jax.lax.rsqrt(vv + 1e-5) * g1e_ref[...] + b1e_ref[...]
    score = jnp.dot(cact, amat_ref[...], preferred_element_type=jnp.float32)
    w8 = jnp.exp(jnp.clip(score, -CLAMP, CLAMP))
    wvec = jnp.dot(w8, sel_ref[...], preferred_element_type=jnp.float32)
    pay_ref[...] = wvec * (vs_ref[...] + jnp.dot(cact, bblk_ref[...],
                                                 preferred_element_type=jnp.float32))
    wden_ref[...] = wvec


def _node_body(x_ref, qh_ref, accp_ref, accd_ref, howt_ref,
               hob_ref, g1h_ref, b1h_ref, w1t_ref, b1_ref, w2t_ref, b2_ref,
               g2h_ref, b2h_ref, h_ref):
    den_b = accd_ref[...]
    attn = jnp.where(den_b > 0.0, accp_ref[...] / den_b, 0.0)
    h_attn = qh_ref[...] + attn
    x = x_ref[...]
    h = x + jnp.dot(h_attn, howt_ref[...],
                    preferred_element_type=jnp.float32) + hob_ref[...]
    m = jnp.mean(h, axis=-1, keepdims=True)
    vv = jnp.mean((h - m) ** 2, axis=-1, keepdims=True)
    h = (h - m) * jax.lax.rsqrt(vv + 1e-5) * g1h_ref[...] + b1h_ref[...]
    h1 = jnp.maximum(jnp.dot(h, w1t_ref[...],
                             preferred_element_type=jnp.float32) + b1_ref[...], 0.0)
    h2 = jnp.dot(h1, w2t_ref[...], preferred_element_type=jnp.float32) + b2_ref[...]
    h = h + h2
    m = jnp.mean(h, axis=-1, keepdims=True)
    vv = jnp.mean((h - m) ** 2, axis=-1, keepdims=True)
    h_ref[...] = (h - m) * jax.lax.rsqrt(vv + 1e-5) * g2h_ref[...] + b2h_ref[...]


def _gather_body(qh, kh, vh, dstv, srcv, g_out, vs_out,
                 dbuf, sbuf, qbuf, kbuf, vbuf, sem1, sem2, sem3):
    c = lax.axis_index("c")
    s = lax.axis_index("s")
    wid = s * NSC + c

    def chunk_body(i, carry):
        g = wid + i * NW

        @pl.when(g < NCHUNK)
        def _():
            base = g * CHUNK
            pltpu.sync_copy(dstv.at[pl.ds(base, CHUNK)], dbuf)
            pltpu.sync_copy(srcv.at[pl.ds(base, CHUNK)], sbuf)
            cp1 = pltpu.async_copy(qh.at[dbuf], qbuf, sem1)
            cp2 = pltpu.async_copy(kh.at[sbuf], kbuf, sem2)
            cp3 = pltpu.async_copy(vh.at[sbuf], vbuf, sem3)
            cp1.wait()
            cp2.wait()

            def row_body(r, carry2):
                for j in range(D // 16):
                    sl = pl.ds(j * 16, 16)
                    qbuf[r, sl] = qbuf[r, sl] + kbuf[r, sl]
                return carry2

            lax.fori_loop(0, CHUNK, row_body, 0)
            pltpu.sync_copy(qbuf, g_out.at[pl.ds(base, CHUNK)])
            cp3.wait()
            pltpu.sync_copy(vbuf, vs_out.at[pl.ds(base, CHUNK)])

        return carry

    lax.fori_loop(0, (NCHUNK + NW - 1) // NW, chunk_body, 0)


def _scatter_body(pay, wden, dstv, accp_out, accd_out,
                  dstbuf, idxbuf, pbuf, dbuf, accp, accd, semp, semd):
    c = lax.axis_index("c")
    s = lax.axis_index("s")
    base_node = c * ROWS_PER_SC

    zeros16 = jnp.zeros((16,), jnp.float32)

    def zrow(r, carry):
        for j in range(D // 16):
            pbuf[r, pl.ds(j * 16, 16)] = zeros16
        return carry

    lax.fori_loop(0, CHUNK, zrow, 0)
    r0 = s * ROWS_PER_TILE
    for acc in (accp, accd):
        pltpu.sync_copy(pbuf, acc.at[pl.ds(r0, CHUNK)])
        pltpu.sync_copy(pbuf, acc.at[pl.ds(r0 + CHUNK, CHUNK)])
        pltpu.sync_copy(pbuf.at[pl.ds(0, ROWS_PER_TILE - 2 * CHUNK)],
                        acc.at[pl.ds(r0 + 2 * CHUNK, ROWS_PER_TILE - 2 * CHUNK)])
    plsc.subcore_barrier()

    def chunk_body(i, carry):
        g = s + i * NTILE

        @pl.when(g < NCHUNK)
        def _():
            base = g * CHUNK
            pltpu.sync_copy(dstv.at[pl.ds(base, CHUNK)], dstbuf)
            for j in range(CHUNK // 16):
                sl = pl.ds(j * 16, 16)
                dv = dstbuf[sl]
                local = dv - base_node
                ok = (local >= 0) & (local < ROWS_PER_SC)
                dummy = ROWS_PER_SC + (dv & 15)
                idxbuf[sl] = jnp.where(ok, local, dummy)
            cpp = pltpu.async_copy(pay.at[pl.ds(base, CHUNK)], pbuf, semp)
            cpd = pltpu.async_copy(wden.at[pl.ds(base, CHUNK)], dbuf, semd)
            cpp.wait()
            pltpu.sync_copy(pbuf, accp.at[idxbuf], add=True)
            cpd.wait()
            pltpu.sync_copy(dbuf, accd.at[idxbuf], add=True)

        return carry

    lax.fori_loop(0, (NCHUNK + NTILE - 1) // NTILE, chunk_body, 0)
    plsc.subcore_barrier()

    o0 = s * OUT_PER_TILE
    g0 = base_node + o0
    for acc, out in ((accp, accp_out), (accd, accd_out)):
        pltpu.sync_copy(acc.at[pl.ds(o0, CHUNK)], out.at[pl.ds(g0, CHUNK)])
        pltpu.sync_copy(acc.at[pl.ds(o0 + CHUNK, CHUNK)],
                        out.at[pl.ds(g0 + CHUNK, CHUNK)])
        pltpu.sync_copy(acc.at[pl.ds(o0 + 2 * CHUNK, OUT_PER_TILE - 2 * CHUNK)],
                        out.at[pl.ds(g0 + 2 * CHUNK, OUT_PER_TILE - 2 * CHUNK)])


def _sc_mesh():
    return plsc.VectorSubcoreMesh(core_axis_name="c", subcore_axis_name="s")


def kernel(x, edge_index, conn, Wq, Wk, Wv, We, Aw, Bw, Ho_w, Ho_b, Eow, Eob,
           g1h, b1h, g1e, b1e, W1, b1, W2, b2, g2h, b2h):
    f32 = jnp.float32
    dst = edge_index[0]
    src = edge_index[1]

    wqkv = jnp.concatenate([Wq.T, Wk.T, Wv.T], axis=1)
    wet = We.T
    eowt = Eow.T
    howt = Ho_w.T
    w1t = W1.T
    w2t = W2.T
    a1 = Aw[:, :, 0].T
    eye8 = jnp.eye(H, dtype=f32)
    amat = (a1[:, :, None] * eye8[:, None, :]).reshape(D, H)
    sel = (eye8[:, :, None] * jnp.ones((1, 1, DH), f32)).reshape(H, D)
    bblk = (Bw.transpose(1, 0, 2)[:, None, :, :]
            * eye8[:, :, None, None]).transpose(0, 2, 1, 3).reshape(D, D)

    row = lambda v: v.reshape(1, -1)

    qh, kh, vh = pl.pallas_call(
        _qkv_body,
        grid=(N // BN,),
        in_specs=[pl.BlockSpec((BN, D), lambda i: (i, 0)),
                  pl.BlockSpec((D, 3 * D), lambda i: (0, 0))],
        out_specs=[pl.BlockSpec((BN, D), lambda i: (i, 0))] * 3,
        out_shape=[jax.ShapeDtypeStruct((N, D), f32)] * 3,
    )(x, wqkv)

    gsum, vs = pl.kernel(
        _gather_body,
        out_type=(jax.ShapeDtypeStruct((E, D), f32),
                  jax.ShapeDtypeStruct((E, D), f32)),
        mesh=_sc_mesh(),
        scratch_types=[
            pltpu.VMEM((CHUNK,), jnp.int32),
            pltpu.VMEM((CHUNK,), jnp.int32),
            pltpu.VMEM((CHUNK, D), f32),
            pltpu.VMEM((CHUNK, D), f32),
            pltpu.VMEM((CHUNK, D), f32),
            pltpu.SemaphoreType.DMA,
            pltpu.SemaphoreType.DMA,
            pltpu.SemaphoreType.DMA,
        ],
    )(qh, kh, vh, dst, src)

    wspec = lambda shape: pl.BlockSpec(shape, lambda i: (0, 0))
    e_out, pay, wden = pl.pallas_call(
        _edge_body,
        grid=(E // BE,),
        in_specs=[pl.BlockSpec((BE, D), lambda i: (i, 0)),
                  pl.BlockSpec((BE, D), lambda i: (i, 0)),
                  pl.BlockSpec((BE, D), lambda i: (i, 0)),
                  wspec((D, 2 * D)), wspec((D, D)), wspec((1, D)),
                  wspec((1, D)), wspec((1, D)), wspec((D, H)),
                  wspec((H, D)), wspec((D, D))],
        out_specs=[pl.BlockSpec((BE, D), lambda i: (i, 0))] * 3,
        out_shape=[jax.ShapeDtypeStruct((E, D), f32)] * 3,
    )(conn, gsum, vs, wet, eowt, row(Eob), row(g1e), row(b1e),
      amat, sel, bblk)

    accp, accd = pl.kernel(
        _scatter_body,
        out_type=(jax.ShapeDtypeStruct((NOUT, D), f32),
                  jax.ShapeDtypeStruct((NOUT, D), f32)),
        mesh=_sc_mesh(),
        scratch_types=[
            pltpu.VMEM((CHUNK,), jnp.int32),
            pltpu.VMEM((CHUNK,), jnp.int32),
            pltpu.VMEM((CHUNK, D), f32),
            pltpu.VMEM((CHUNK, D), f32),
            pltpu.VMEM_SHARED((ACC_ROWS, D), f32),
            pltpu.VMEM_SHARED((ACC_ROWS, D), f32),
            pltpu.SemaphoreType.DMA,
            pltpu.SemaphoreType.DMA,
        ],
    )(pay, wden, dst)

    h = pl.pallas_call(
        _node_body,
        grid=(N // BN,),
        in_specs=[pl.BlockSpec((BN, D), lambda i: (i, 0)),
                  pl.BlockSpec((BN, D), lambda i: (i, 0)),
                  pl.BlockSpec((BN, D), lambda i: (i, 0)),
                  pl.BlockSpec((BN, D), lambda i: (i, 0)),
                  wspec((D, D)), wspec((1, D)),
                  wspec((1, D)), wspec((1, D)), wspec((D, 2 * D)),
                  wspec((1, 2 * D)), wspec((2 * D, D)), wspec((1, D)),
                  wspec((1, D)), wspec((1, D))],
        out_specs=pl.BlockSpec((BN, D), lambda i: (i, 0)),
        out_shape=jax.ShapeDtypeStruct((N, D), f32),
    )(x, qh, accp, accd, howt, row(Ho_b), row(g1h), row(b1h),
      w1t, row(b1), w2t, row(b2), row(g2h), row(b2h))

    return (h, e_out)


# speedup vs baseline: 66.9858x; 1.3900x over previous
import functools

import jax
import jax.numpy as jnp
from jax import lax
from jax.experimental import pallas as pl
from jax.experimental.pallas import tpu as pltpu
from jax.experimental.pallas import tpu_sc as plsc

N = 10000
E = 320000
D = 128
H = 8
DH = 16
CLAMP = 5.0

NSC = 2
NTILE = 16
NW = NSC * NTILE
CHUNK = 128
NCHUNK = E // CHUNK
NOUT = 10240
ROWS_PER_TILE = NOUT // NTILE

BN = 2000
BE = 1280


def _qkv_body(x_ref, w_ref, q_ref, k_ref, v_ref):
    y = jnp.dot(x_ref[...], w_ref[...], preferred_element_type=jnp.float32)
    q_ref[...] = y[:, :D]
    k_ref[...] = y[:, D:2 * D]
    v_ref[...] = y[:, 2 * D:]


def _edge_body(conn_ref, g_ref, vs_ref, wet_ref, eowt_ref, eob_ref, g1e_ref,
               b1e_ref, amat_ref, sel_ref, bblk_ref,
               e_ref, pay_ref, wden_ref):
    conn = conn_ref[...]
    eh = jnp.dot(conn, wet_ref[...], preferred_element_type=jnp.float32)
    ew = eh[:, :D]
    eb = eh[:, D:]
    c1 = g_ref[...] * ew
    c2 = jnp.sign(c1) * jnp.sqrt(jnp.abs(c1))
    cact = jnp.maximum(c2 + eb, 0.0)
    ev = conn + jnp.dot(cact, eowt_ref[...],
                        preferred_element_type=jnp.float32) + eob_ref[...]
    m = jnp.mean(ev, axis=-1, keepdims=True)
    vv = jnp.mean((ev - m) ** 2, axis=-1, keepdims=True)
    e_ref[...] = (ev - m) * jax.lax.rsqrt(vv + 1e-5) * g1e_ref[...] + b1e_ref[...]
    score = jnp.dot(cact, amat_ref[...], preferred_element_type=jnp.float32)
    w8 = jnp.exp(jnp.clip(score, -CLAMP, CLAMP))
    wvec = jnp.dot(w8, sel_ref[...], preferred_element_type=jnp.float32)
    pay_ref[...] = wvec * (vs_ref[...] + jnp.dot(cact, bblk_ref[...],
                                                 preferred_element_type=jnp.float32))
    wden_ref[...] = wvec


def _node_body(x_ref, qh_ref, accp_ref, accd_ref, howt_ref,
               hob_ref, g1h_ref, b1h_ref, w1t_ref, b1_ref, w2t_ref, b2_ref,
               g2h_ref, b2h_ref, h_ref):
    den_b = accd_ref[...]
    attn = jnp.where(den_b > 0.0, accp_ref[...] / den_b, 0.0)
    h_attn = qh_ref[...] + attn
    x = x_ref[...]
    h = x + jnp.dot(h_attn, howt_ref[...],
                    preferred_element_type=jnp.float32) + hob_ref[...]
    m = jnp.mean(h, axis=-1, keepdims=True)
    vv = jnp.mean((h - m) ** 2, axis=-1, keepdims=True)
    h = (h - m) * jax.lax.rsqrt(vv + 1e-5) * g1h_ref[...] + b1h_ref[...]
    h1 = jnp.maximum(jnp.dot(h, w1t_ref[...],
                             preferred_element_type=jnp.float32) + b1_ref[...], 0.0)
    h2 = jnp.dot(h1, w2t_ref[...], preferred_element_type=jnp.float32) + b2_ref[...]
    h = h + h2
    m = jnp.mean(h, axis=-1, keepdims=True)
    vv = jnp.mean((h - m) ** 2, axis=-1, keepdims=True)
    h_ref[...] = (h - m) * jax.lax.rsqrt(vv + 1e-5) * g2h_ref[...] + b2h_ref[...]


def _gather_body(qh, kh, vh, dstv, srcv, g_out, vs_out,
                 da, sa, db, sb, qa, ka, va, qb, kb, vb,
                 sda, ssa, sqa, ska, sva, sdb, ssb, sqb, skb, svb):
    c = lax.axis_index("c")
    s = lax.axis_index("s")
    wid = s * NSC + c
    NI = (NCHUNK + NW - 1) // NW

    def start_idx(d, s_, sd, ss, i):
        g = wid + i * NW

        @pl.when(g < NCHUNK)
        def _():
            base = g * CHUNK
            pltpu.async_copy(dstv.at[pl.ds(base, CHUNK)], d, sd)
            pltpu.async_copy(srcv.at[pl.ds(base, CHUNK)], s_, ss)

    def start_gather(d, s_, q, k, v, sd, ss, sq, sk, sv, i):
        g = wid + i * NW

        @pl.when(g < NCHUNK)
        def _():
            base = g * CHUNK
            pltpu.make_async_copy(dstv.at[pl.ds(base, CHUNK)], d, sd).wait()
            pltpu.make_async_copy(srcv.at[pl.ds(base, CHUNK)], s_, ss).wait()
            pltpu.async_copy(qh.at[d], q, sq)
            pltpu.async_copy(kh.at[s_], k, sk)
            pltpu.async_copy(vh.at[s_], v, sv)

    def process(d, s_, q, k, v, sq, sk, sv, i):
        g = wid + i * NW

        @pl.when(g < NCHUNK)
        def _():
            base = g * CHUNK
            pltpu.make_async_copy(qh.at[d], q, sq).wait()
            pltpu.make_async_copy(kh.at[s_], k, sk).wait()

            def row_body(r, carry2):
                for j in range(D // 16):
                    sl = pl.ds(j * 16, 16)
                    q[r, sl] = q[r, sl] + k[r, sl]
                return carry2

            lax.fori_loop(0, CHUNK, row_body, 0)
            pltpu.sync_copy(q, g_out.at[pl.ds(base, CHUNK)])
            pltpu.make_async_copy(vh.at[s_], v, sv).wait()
            pltpu.sync_copy(v, vs_out.at[pl.ds(base, CHUNK)])

    A = (da, sa, qa, ka, va, sda, ssa, sqa, ska, sva)
    B = (db, sb, qb, kb, vb, sdb, ssb, sqb, skb, svb)

    def g_start(S, i):
        start_gather(S[0], S[1], S[2], S[3], S[4], S[5], S[6], S[7], S[8], S[9], i)

    def g_proc(S, i):
        process(S[0], S[1], S[2], S[3], S[4], S[7], S[8], S[9], i)

    start_idx(A[0], A[1], A[5], A[6], 0)

    def pair_body(ii, carry):
        i0 = 2 * ii
        g_start(A, i0)
        start_idx(B[0], B[1], B[5], B[6], i0 + 1)
        g_proc(A, i0)
        g_start(B, i0 + 1)
        start_idx(A[0], A[1], A[5], A[6], i0 + 2)
        g_proc(B, i0 + 1)
        return carry

    lax.fori_loop(0, (NI + 1) // 2, pair_body, 0)


def _scatter_body(pay, wden, dstv, accp_out, accd_out,
                  dstbuf_a, dstbuf_b, pbuf_a, pbuf_b, accu,
                  sda, sdb, spa, spb):
    c = lax.axis_index("c")
    s = lax.axis_index("s")

    zeros16 = jnp.zeros((16,), jnp.float32)

    def zrow(r, carry):
        for j in range(D // 16):
            pbuf_a[r, pl.ds(j * 16, 16)] = zeros16
        return carry

    lax.fori_loop(0, CHUNK, zrow, 0)
    r0 = s * ROWS_PER_TILE
    for k in range(ROWS_PER_TILE // CHUNK):
        pltpu.sync_copy(pbuf_a, accu.at[pl.ds(r0 + k * CHUNK, CHUNK)])
    plsc.subcore_barrier()

    NI = (NCHUNK + NTILE - 1) // NTILE

    def _run(src):
        def start(dbuf, pbuf, sd, sp, i):
            g = s + i * NTILE

            @pl.when(g < NCHUNK)
            def _():
                base = g * CHUNK
                pltpu.async_copy(dstv.at[pl.ds(base, CHUNK)], dbuf, sd)
                pltpu.async_copy(src.at[pl.ds(base, CHUNK)], pbuf, sp)

        def finish(dbuf, pbuf, sd, sp, i):
            g = s + i * NTILE

            @pl.when(g < NCHUNK)
            def _():
                base = g * CHUNK
                pltpu.make_async_copy(dstv.at[pl.ds(base, CHUNK)], dbuf, sd).wait()
                pltpu.make_async_copy(src.at[pl.ds(base, CHUNK)], pbuf, sp).wait()
                pltpu.sync_copy(pbuf, accu.at[dbuf], add=True)

        start(dstbuf_a, pbuf_a, sda, spa, 0)

        def pair_body(ii, carry):
            i0 = 2 * ii
            start(dstbuf_b, pbuf_b, sdb, spb, i0 + 1)
            finish(dstbuf_a, pbuf_a, sda, spa, i0)
            start(dstbuf_a, pbuf_a, sda, spa, i0 + 2)
            finish(dstbuf_b, pbuf_b, sdb, spb, i0 + 1)
            return carry

        lax.fori_loop(0, (NI + 1) // 2, pair_body, 0)

    @pl.when(c == 0)
    def _():
        _run(pay)

    @pl.when(c == 1)
    def _():
        _run(wden)

    plsc.subcore_barrier()

    def _wout(out):
        for k in range(ROWS_PER_TILE // CHUNK):
            pltpu.sync_copy(accu.at[pl.ds(r0 + k * CHUNK, CHUNK)],
                            out.at[pl.ds(r0 + k * CHUNK, CHUNK)])

    @pl.when(c == 0)
    def _():
        _wout(accp_out)

    @pl.when(c == 1)
    def _():
        _wout(accd_out)


def _sc_mesh():
    return plsc.VectorSubcoreMesh(core_axis_name="c", subcore_axis_name="s")


def kernel(x, edge_index, conn, Wq, Wk, Wv, We, Aw, Bw, Ho_w, Ho_b, Eow, Eob,
           g1h, b1h, g1e, b1e, W1, b1, W2, b2, g2h, b2h):
    f32 = jnp.float32
    dst = edge_index[0]
    src = edge_index[1]

    wqkv = jnp.concatenate([Wq.T, Wk.T, Wv.T], axis=1)
    wet = We.T
    eowt = Eow.T
    howt = Ho_w.T
    w1t = W1.T
    w2t = W2.T
    a1 = Aw[:, :, 0].T
    eye8 = jnp.eye(H, dtype=f32)
    amat = (a1[:, :, None] * eye8[:, None, :]).reshape(D, H)
    sel = (eye8[:, :, None] * jnp.ones((1, 1, DH), f32)).reshape(H, D)
    bblk = (Bw.transpose(1, 0, 2)[:, None, :, :]
            * eye8[:, :, None, None]).transpose(0, 2, 1, 3).reshape(D, D)

    row = lambda v: v.reshape(1, -1)

    qh, kh, vh = pl.pallas_call(
        _qkv_body,
        grid=(N // BN,),
        in_specs=[pl.BlockSpec((BN, D), lambda i: (i, 0)),
                  pl.BlockSpec((D, 3 * D), lambda i: (0, 0))],
        out_specs=[pl.BlockSpec((BN, D), lambda i: (i, 0))] * 3,
        out_shape=[jax.ShapeDtypeStruct((N, D), f32)] * 3,
    )(x, wqkv)

    gsum, vs = pl.kernel(
        _gather_body,
        out_type=(jax.ShapeDtypeStruct((E, D), f32),
                  jax.ShapeDtypeStruct((E, D), f32)),
        mesh=_sc_mesh(),
        scratch_types=(
            [pltpu.VMEM((CHUNK,), jnp.int32)] * 2
            + [pltpu.VMEM((CHUNK,), jnp.int32)] * 2
            + [pltpu.VMEM((CHUNK, D), f32)] * 3
            + [pltpu.VMEM((CHUNK, D), f32)] * 3
            + [pltpu.SemaphoreType.DMA] * 10
        ),
    )(qh, kh, vh, dst, src)

    wspec = lambda shape: pl.BlockSpec(shape, lambda i: (0, 0))
    e_out, pay, wden = pl.pallas_call(
        _edge_body,
        grid=(E // BE,),
        in_specs=[pl.BlockSpec((BE, D), lambda i: (i, 0)),
                  pl.BlockSpec((BE, D), lambda i: (i, 0)),
                  pl.BlockSpec((BE, D), lambda i: (i, 0)),
                  wspec((D, 2 * D)), wspec((D, D)), wspec((1, D)),
                  wspec((1, D)), wspec((1, D)), wspec((D, H)),
                  wspec((H, D)), wspec((D, D))],
        out_specs=[pl.BlockSpec((BE, D), lambda i: (i, 0))] * 3,
        out_shape=[jax.ShapeDtypeStruct((E, D), f32)] * 3,
    )(conn, gsum, vs, wet, eowt, row(Eob), row(g1e), row(b1e),
      amat, sel, bblk)

    accp, accd = pl.kernel(
        _scatter_body,
        out_type=(jax.ShapeDtypeStruct((NOUT, D), f32),
                  jax.ShapeDtypeStruct((NOUT, D), f32)),
        mesh=_sc_mesh(),
        scratch_types=[
            pltpu.VMEM((CHUNK,), jnp.int32),
            pltpu.VMEM((CHUNK,), jnp.int32),
            pltpu.VMEM((CHUNK, D), f32),
            pltpu.VMEM((CHUNK, D), f32),
            pltpu.VMEM_SHARED((NOUT, D), f32),
            pltpu.SemaphoreType.DMA,
            pltpu.SemaphoreType.DMA,
            pltpu.SemaphoreType.DMA,
            pltpu.SemaphoreType.DMA,
        ],
    )(pay, wden, dst)

    h = pl.pallas_call(
        _node_body,
        grid=(N // BN,),
        in_specs=[pl.BlockSpec((BN, D), lambda i: (i, 0)),
                  pl.BlockSpec((BN, D), lambda i: (i, 0)),
                  pl.BlockSpec((BN, D), lambda i: (i, 0)),
                  pl.BlockSpec((BN, D), lambda i: (i, 0)),
                  wspec((D, D)), wspec((1, D)),
                  wspec((1, D)), wspec((1, D)), wspec((D, 2 * D)),
                  wspec((1, 2 * D)), wspec((2 * D, D)), wspec((1, D)),
                  wspec((1, D)), wspec((1, D))],
        out_specs=pl.BlockSpec((BN, D), lambda i: (i, 0)),
        out_shape=jax.ShapeDtypeStruct((N, D), f32),
    )(x, qh, accp, accd, howt, row(Ho_b), row(g1h), row(b1h),
      w1t, row(b1), w2t, row(b2), row(g2h), row(b2h))

    return (h, e_out)
